# baseline (speedup 1.0000x reference)
import functools

import jax
import jax.numpy as jnp
from jax import lax
from jax.experimental import pallas as pl
from jax.experimental.pallas import tpu as pltpu
from jax.experimental.pallas import tpu_sc as plsc

B, N, M = 16, 2048, 64
CUTOFF = 6.0
NUM_WORKERS = 32
ATOMS_PER_W = (B * N) // NUM_WORKERS
CHUNK = 128
NCHUNK = ATOMS_PER_W // CHUNK
GROUPS = CHUNK // 16
UNROLL = 8
PACK_ROWS = 512


def _pack(d_ref, pt_ref, out_ref):
    d = d_ref[0]
    pt = pt_ref[0]
    bits = (lax.bitcast_convert_type(d, jnp.int32) & (-4)) | pt
    fused = lax.bitcast_convert_type(bits, jnp.float32)
    f3 = fused.reshape(PACK_ROWS // 2, 2, M)
    out_ref[...] = jnp.concatenate([f3[:, 0, :], f3[:, 1, :]], axis=1)


def _sc_stage(f_hbm, coef_hbm, phi_hbm, rho_hbm,
              fb_a, fb_b, coef_v, phi_v, rho_v, sem_a, sem_b):
    wid = lax.axis_index("c") * 16 + lax.axis_index("s")
    lane = lax.iota(jnp.int32, 16)
    lane64 = lane * M
    wbase = wid * (ATOMS_PER_W * M // 128)

    pltpu.sync_copy(coef_hbm, coef_v)

    zero16 = jnp.zeros((16,), jnp.float32)

    rows = CHUNK * M // 128

    def start(buf, sem, c):
        src = f_hbm.at[pl.ds(wbase + c * rows, rows)]
        pltpu.make_async_copy(src, buf, sem).start()

    def wait(buf, sem):
        pltpu.make_async_copy(f_hbm.at[pl.ds(0, rows)], buf, sem).wait()

    def compute(buf, c):
        for g in range(GROUPS):
            gbase = lane64 + g * 16 * M

            def body(jj, acc):
                ap, ar = acc
                for u in range(UNROLL):
                    idx = gbase + (jj * UNROLL + u)
                    fused = plsc.load_gather(
                        buf, [lax.shift_right_logical(idx, 7), idx & 127])
                    bits = plsc.bitcast(fused, jnp.int32)
                    pt = bits & 3
                    d = plsc.bitcast(bits & (-4), jnp.float32)
                    ca1 = plsc.load_gather(coef_v, [pt])
                    cb1 = plsc.load_gather(coef_v, [pt + 3])
                    ca2 = plsc.load_gather(coef_v, [pt + 6])
                    cb2 = plsc.load_gather(coef_v, [pt + 9])
                    off = plsc.load_gather(coef_v, [pt + 12])
                    m = d < CUTOFF
                    phi = ca1 * jnp.exp(cb1 * d) - off
                    rho = ca2 * jnp.exp(cb2 * d)
                    ap = ap + jnp.where(m, phi, 0.0)
                    ar = ar + jnp.where(m, rho, 0.0)
                return ap, ar

            ap, ar = lax.fori_loop(0, M // UNROLL, body, (zero16, zero16))
            oidx = c * CHUNK + g * 16 + lane
            plsc.store_scatter(phi_v, [oidx], ap)
            plsc.store_scatter(rho_v, [oidx], ar)

    start(fb_a, sem_a, 0)

    def pair_body(c2, carry):
        c = c2 * 2
        start(fb_b, sem_b, c + 1)
        wait(fb_a, sem_a)
        compute(fb_a, c)
        start(fb_a, sem_a, lax.min(c + 2, NCHUNK - 1))
        wait(fb_b, sem_b)
        compute(fb_b, c + 1)
        return carry

    lax.fori_loop(0, NCHUNK // 2, pair_body, 0)
    wait(fb_a, sem_a)

    pltpu.sync_copy(phi_v, phi_hbm.at[pl.ds(wid * ATOMS_PER_W, ATOMS_PER_W)])
    pltpu.sync_copy(rho_v, rho_hbm.at[pl.ds(wid * ATOMS_PER_W, ATOMS_PER_W)])


def _tc_finale(phi_ref, rho_ref, t_ref, n_ref, f_ref, out_ref):
    phi = phi_ref[...]
    rho = rho_ref[...]
    t = t_ref[...]
    f0 = f_ref[0, 0]
    f1 = f_ref[1, 0]
    fs = jnp.where(t == 0, f0, f1)
    emb = -fs * jnp.sqrt(rho + 1e-12)
    col = lax.broadcasted_iota(jnp.int32, (B, N), 1)
    n = n_ref[:, 0:1]
    amask = col < n
    ae = jnp.where(amask, 0.5 * phi + emb, 0.0)
    e = jnp.sum(ae, axis=1, keepdims=True)
    epa = e / n.astype(jnp.float32)
    out_ref[...] = jnp.broadcast_to(epa, (B, 128))


def kernel(types, pair_types, distances, n_atoms, A, p, q, xi, r0, F_scale,
           offsets):
    f32 = jnp.float32
    ca1 = (A * jnp.exp(p)).astype(f32)
    cb1 = (-p / r0).astype(f32)
    ca2 = (xi * xi * jnp.exp(2.0 * q)).astype(f32)
    cb2 = (-2.0 * q / r0).astype(f32)
    coef = jnp.concatenate(
        [ca1, cb1, ca2, cb2, offsets.astype(f32), jnp.zeros((1,), f32)])

    f1 = pl.pallas_call(
        _pack,
        grid=(B, N // PACK_ROWS),
        in_specs=[
            pl.BlockSpec((1, PACK_ROWS, M), lambda i, j: (i, j, 0)),
            pl.BlockSpec((1, PACK_ROWS, M), lambda i, j: (i, j, 0)),
        ],
        out_specs=pl.BlockSpec((PACK_ROWS * M // 128, 128),
                               lambda i, j: (i * (N // PACK_ROWS) + j, 0)),
        out_shape=jax.ShapeDtypeStruct((B * N * M // 128, 128), f32),
    )(distances, pair_types)

    mesh = plsc.VectorSubcoreMesh(core_axis_name="c", subcore_axis_name="s")
    sc = functools.partial(
        pl.kernel,
        mesh=mesh,
        compiler_params=pltpu.CompilerParams(needs_layout_passes=False),
        out_type=[
            jax.ShapeDtypeStruct((B * N,), f32),
            jax.ShapeDtypeStruct((B * N,), f32),
        ],
        scratch_types=[
            pltpu.VMEM((CHUNK * M // 128, 128), f32),
            pltpu.VMEM((CHUNK * M // 128, 128), f32),
            pltpu.VMEM((16,), f32),
            pltpu.VMEM((ATOMS_PER_W,), f32),
            pltpu.VMEM((ATOMS_PER_W,), f32),
            pltpu.SemaphoreType.DMA,
            pltpu.SemaphoreType.DMA,
        ],
    )(_sc_stage)
    phi_sum, rho_sum = sc(f1, coef)

    phi2 = phi_sum.reshape(B, N)
    rho2 = rho_sum.reshape(B, N)
    nb = jnp.broadcast_to(n_atoms.reshape(B, 1), (B, 128)).astype(jnp.int32)
    fpad = jnp.concatenate(
        [jnp.broadcast_to(F_scale.reshape(2, 1).astype(f32), (2, 128)),
         jnp.zeros((6, 128), f32)])

    out = pl.pallas_call(
        _tc_finale,
        out_shape=jax.ShapeDtypeStruct((B, 128), f32),
    )(phi2, rho2, types, nb, fpad)
    return out[:, :1]


# speedup vs baseline: 1.3433x; 1.3433x over previous
import functools

import jax
import jax.numpy as jnp
from jax import lax
from jax.experimental import pallas as pl
from jax.experimental.pallas import tpu as pltpu
from jax.experimental.pallas import tpu_sc as plsc

B, N, M = 16, 2048, 64
CUTOFF = 6.0
NUM_WORKERS = 32
ATOMS_PER_W = (B * N) // NUM_WORKERS
CHUNK = 128
NCHUNK = ATOMS_PER_W // CHUNK
GROUPS = CHUNK // 16
UNROLL = 8
PACK_ROWS = 512


def _sc_stage(f_hbm, coef_hbm, phi_hbm, rho_hbm,
              fb_a, fb_b, coef_v, phi_v, rho_v, sem_a, sem_b):
    wid = lax.axis_index("c") * 16 + lax.axis_index("s")
    lane = lax.iota(jnp.int32, 16)
    lane64 = lane * M
    wbase = wid * ATOMS_PER_W * M

    pltpu.sync_copy(coef_hbm, coef_v)

    zero16 = jnp.zeros((16,), jnp.float32)

    def start(buf, sem, c):
        src = f_hbm.at[pl.ds(wbase + c * CHUNK * M, CHUNK * M)]
        pltpu.make_async_copy(src, buf, sem).start()

    def wait(buf, sem):
        pltpu.make_async_copy(f_hbm.at[pl.ds(0, CHUNK * M)], buf, sem).wait()

    def compute(buf, c):
        for g in range(GROUPS):
            gbase = lane64 + g * 16 * M

            def body(jj, acc):
                ap, ar = acc
                for u in range(UNROLL):
                    idx = gbase + (jj * UNROLL + u)
                    fused = plsc.load_gather(buf, [idx])
                    bits = plsc.bitcast(fused, jnp.int32)
                    pt = bits & 3
                    d = plsc.bitcast(bits & (-4), jnp.float32)
                    ca1 = plsc.load_gather(coef_v, [pt])
                    cb1 = plsc.load_gather(coef_v, [pt + 3])
                    ca2 = plsc.load_gather(coef_v, [pt + 6])
                    cb2 = plsc.load_gather(coef_v, [pt + 9])
                    off = plsc.load_gather(coef_v, [pt + 12])
                    m = d < CUTOFF
                    phi = ca1 * jnp.exp(cb1 * d) - off
                    rho = ca2 * jnp.exp(cb2 * d)
                    ap = ap + jnp.where(m, phi, 0.0)
                    ar = ar + jnp.where(m, rho, 0.0)
                return ap, ar

            ap, ar = lax.fori_loop(0, M // UNROLL, body, (zero16, zero16))
            oidx = c * CHUNK + g * 16 + lane
            plsc.store_scatter(phi_v, [oidx], ap)
            plsc.store_scatter(rho_v, [oidx], ar)

    start(fb_a, sem_a, 0)

    def pair_body(c2, carry):
        c = c2 * 2
        start(fb_b, sem_b, c + 1)
        wait(fb_a, sem_a)
        compute(fb_a, c)
        start(fb_a, sem_a, lax.min(c + 2, NCHUNK - 1))
        wait(fb_b, sem_b)
        compute(fb_b, c + 1)
        return carry

    lax.fori_loop(0, NCHUNK // 2, pair_body, 0)
    wait(fb_a, sem_a)

    pltpu.sync_copy(phi_v, phi_hbm.at[pl.ds(wid * ATOMS_PER_W, ATOMS_PER_W)])
    pltpu.sync_copy(rho_v, rho_hbm.at[pl.ds(wid * ATOMS_PER_W, ATOMS_PER_W)])


def _tc_finale(phi_ref, rho_ref, t_ref, n_ref, f_ref, out_ref):
    phi = phi_ref[...]
    rho = rho_ref[...]
    t = t_ref[...]
    f0 = f_ref[0, 0]
    f1 = f_ref[1, 0]
    fs = jnp.where(t == 0, f0, f1)
    emb = -fs * jnp.sqrt(rho + 1e-12)
    col = lax.broadcasted_iota(jnp.int32, (B, N), 1)
    n = n_ref[:, 0:1]
    amask = col < n
    ae = jnp.where(amask, 0.5 * phi + emb, 0.0)
    e = jnp.sum(ae, axis=1, keepdims=True)
    epa = e / n.astype(jnp.float32)
    out_ref[...] = jnp.broadcast_to(epa, (B, 128))


def kernel(types, pair_types, distances, n_atoms, A, p, q, xi, r0, F_scale,
           offsets):
    f32 = jnp.float32
    ca1 = (A * jnp.exp(p)).astype(f32)
    cb1 = (-p / r0).astype(f32)
    ca2 = (xi * xi * jnp.exp(2.0 * q)).astype(f32)
    cb2 = (-2.0 * q / r0).astype(f32)
    coef = jnp.concatenate(
        [ca1, cb1, ca2, cb2, offsets.astype(f32), jnp.zeros((1,), f32)])

    dbits = lax.bitcast_convert_type(distances.reshape(-1), jnp.int32)
    f1 = lax.bitcast_convert_type(
        (dbits & (-4)) | pair_types.reshape(-1), f32)

    mesh = plsc.VectorSubcoreMesh(core_axis_name="c", subcore_axis_name="s")
    sc = functools.partial(
        pl.kernel,
        mesh=mesh,
        compiler_params=pltpu.CompilerParams(needs_layout_passes=False),
        out_type=[
            jax.ShapeDtypeStruct((B * N,), f32),
            jax.ShapeDtypeStruct((B * N,), f32),
        ],
        scratch_types=[
            pltpu.VMEM((CHUNK * M,), f32),
            pltpu.VMEM((CHUNK * M,), f32),
            pltpu.VMEM((16,), f32),
            pltpu.VMEM((ATOMS_PER_W,), f32),
            pltpu.VMEM((ATOMS_PER_W,), f32),
            pltpu.SemaphoreType.DMA,
            pltpu.SemaphoreType.DMA,
        ],
    )(_sc_stage)
    phi_sum, rho_sum = sc(f1, coef)

    phi2 = phi_sum.reshape(B, N)
    rho2 = rho_sum.reshape(B, N)
    nb = jnp.broadcast_to(n_atoms.reshape(B, 1), (B, 128)).astype(jnp.int32)
    fpad = jnp.concatenate(
        [jnp.broadcast_to(F_scale.reshape(2, 1).astype(f32), (2, 128)),
         jnp.zeros((6, 128), f32)])

    out = pl.pallas_call(
        _tc_finale,
        out_shape=jax.ShapeDtypeStruct((B, 128), f32),
    )(phi2, rho2, types, nb, fpad)
    return out[:, :1]
